# Initial kernel scaffold; baseline (speedup 1.0000x reference)
import jax
import jax.numpy as jnp
from jax.experimental import pallas as pl

B, G, IN_C, OUT_C, K = 8, 2048, 256, 256, 16
TG = 256


def _lnp_kernel(xyz_rows_ref, xyz_cols_ref, feat_ref, w1_ref, b1_ref,
                w2_ref, b2_ref, out_ref):
    xr = xyz_rows_ref[0]
    xc = xyz_cols_ref[0]
    sqr = jnp.sum(xr * xr, axis=1, keepdims=True)
    sqc = jnp.sum(xc * xc, axis=0, keepdims=True)
    dot = jax.lax.dot_general(
        xr, xc, (((1,), (0,)), ((), ())),
        preferred_element_type=jnp.float32,
        precision=jax.lax.Precision.HIGHEST)
    d2 = jnp.maximum(sqr + sqc - 2.0 * dot, 0.0)

    iota = jax.lax.broadcasted_iota(jnp.int32, (TG, G), 1)
    big_idx = jnp.int32(G)
    inf = jnp.float32(3.4e38)

    def body(_, carry):
        d2c, mask = carry
        v = jnp.min(d2c, axis=1, keepdims=True)
        cand = jnp.where(d2c <= v, iota, big_idx)
        first = jnp.min(cand, axis=1, keepdims=True)
        onehot = iota == first
        mask = mask + onehot.astype(jnp.float32)
        d2c = jnp.where(onehot, inf, d2c)
        return d2c, mask

    mask0 = jnp.zeros((TG, G), dtype=jnp.float32)
    _, mask = jax.lax.fori_loop(0, K, body, (d2, mask0))

    feat = feat_ref[0]
    featsq = feat * feat
    pooled_sq = jax.lax.dot_general(
        mask, featsq, (((1,), (0,)), ((), ())),
        preferred_element_type=jnp.float32,
        precision=jax.lax.Precision.HIGHEST)
    pooled = jnp.sqrt(pooled_sq)

    h = jax.lax.dot_general(
        pooled, w1_ref[...], (((1,), (0,)), ((), ())),
        preferred_element_type=jnp.float32,
        precision=jax.lax.Precision.HIGHEST) + b1_ref[...]
    h = jax.nn.gelu(h, approximate=False)
    out = jax.lax.dot_general(
        h, w2_ref[...], (((1,), (0,)), ((), ())),
        preferred_element_type=jnp.float32,
        precision=jax.lax.Precision.HIGHEST) + b2_ref[...]
    out_ref[0] = out


@jax.jit
def kernel(xyz, feat, W1, b1, W2, b2):
    xyzT = jnp.transpose(xyz, (0, 2, 1))
    w1t = W1.T
    w2t = W2.T
    b1r = b1.reshape(1, OUT_C)
    b2r = b2.reshape(1, OUT_C)

    grid = (B, G // TG)
    return pl.pallas_call(
        _lnp_kernel,
        grid=grid,
        in_specs=[
            pl.BlockSpec((1, TG, 3), lambda b, i: (b, i, 0)),
            pl.BlockSpec((1, 3, G), lambda b, i: (b, 0, 0)),
            pl.BlockSpec((1, G, IN_C), lambda b, i: (b, 0, 0)),
            pl.BlockSpec((IN_C, OUT_C), lambda b, i: (0, 0)),
            pl.BlockSpec((1, OUT_C), lambda b, i: (0, 0)),
            pl.BlockSpec((OUT_C, OUT_C), lambda b, i: (0, 0)),
            pl.BlockSpec((1, OUT_C), lambda b, i: (0, 0)),
        ],
        out_specs=pl.BlockSpec((1, TG, OUT_C), lambda b, i: (b, i, 0)),
        out_shape=jax.ShapeDtypeStruct((B, G, OUT_C), jnp.float32),
    )(xyz, xyzT, feat, w1t, b1r, w2t, b2r)


# speedup vs baseline: 6.9078x; 6.9078x over previous
import jax
import jax.numpy as jnp
from jax.experimental import pallas as pl

B, G, IN_C, OUT_C, K = 8, 2048, 256, 256, 16
TG = 256


def _lnp_kernel(xyz_rows_ref, xyz_cols_ref, feat_ref, w1_ref, b1_ref,
                w2_ref, b2_ref, out_ref):
    xr = xyz_rows_ref[0]
    xc = xyz_cols_ref[0]
    sqr = jnp.sum(xr * xr, axis=1, keepdims=True)
    sqc = jnp.sum(xc * xc, axis=0, keepdims=True)
    dot = jax.lax.dot_general(
        xr, xc, (((1,), (0,)), ((), ())),
        preferred_element_type=jnp.float32,
        precision=jax.lax.Precision.DEFAULT)
    d2 = jnp.maximum(sqr + sqc - 2.0 * dot, 0.0)

    iota = jax.lax.broadcasted_iota(jnp.int32, (TG, G), 1)
    big_idx = jnp.int32(G)
    inf = jnp.float32(3.4e38)

    def body(_, carry):
        d2c, mask = carry
        v = jnp.min(d2c, axis=1, keepdims=True)
        cand = jnp.where(d2c <= v, iota, big_idx)
        first = jnp.min(cand, axis=1, keepdims=True)
        onehot = iota == first
        mask = mask + onehot.astype(jnp.float32)
        d2c = jnp.where(onehot, inf, d2c)
        return d2c, mask

    mask0 = jnp.zeros((TG, G), dtype=jnp.float32)
    _, mask = jax.lax.fori_loop(0, K, body, (d2, mask0))

    feat = feat_ref[0]
    featsq = feat * feat
    pooled_sq = jax.lax.dot_general(
        mask, featsq, (((1,), (0,)), ((), ())),
        preferred_element_type=jnp.float32,
        precision=jax.lax.Precision.DEFAULT)
    pooled = jnp.sqrt(pooled_sq)

    h = jax.lax.dot_general(
        pooled, w1_ref[...], (((1,), (0,)), ((), ())),
        preferred_element_type=jnp.float32,
        precision=jax.lax.Precision.DEFAULT) + b1_ref[...]
    h = 0.5 * h * (1.0 + jax.lax.erf(h * jnp.float32(0.7071067811865476)))
    out = jax.lax.dot_general(
        h, w2_ref[...], (((1,), (0,)), ((), ())),
        preferred_element_type=jnp.float32,
        precision=jax.lax.Precision.DEFAULT) + b2_ref[...]
    out_ref[0] = out


@jax.jit
def kernel(xyz, feat, W1, b1, W2, b2):
    xyzT = jnp.transpose(xyz, (0, 2, 1))
    w1t = W1.T
    w2t = W2.T
    b1r = b1.reshape(1, OUT_C)
    b2r = b2.reshape(1, OUT_C)

    grid = (B, G // TG)
    return pl.pallas_call(
        _lnp_kernel,
        grid=grid,
        in_specs=[
            pl.BlockSpec((1, TG, 3), lambda b, i: (b, i, 0)),
            pl.BlockSpec((1, 3, G), lambda b, i: (b, 0, 0)),
            pl.BlockSpec((1, G, IN_C), lambda b, i: (b, 0, 0)),
            pl.BlockSpec((IN_C, OUT_C), lambda b, i: (0, 0)),
            pl.BlockSpec((1, OUT_C), lambda b, i: (0, 0)),
            pl.BlockSpec((OUT_C, OUT_C), lambda b, i: (0, 0)),
            pl.BlockSpec((1, OUT_C), lambda b, i: (0, 0)),
        ],
        out_specs=pl.BlockSpec((1, TG, OUT_C), lambda b, i: (b, i, 0)),
        out_shape=jax.ShapeDtypeStruct((B, G, OUT_C), jnp.float32),
    )(xyz, xyzT, feat, w1t, b1r, w2t, b2r)


# speedup vs baseline: 26.7097x; 3.8666x over previous
import jax
import jax.numpy as jnp
from jax.experimental import pallas as pl

B, G, IN_C, OUT_C, K = 8, 2048, 256, 256, 16
TG = 256


def _lnp_kernel(xyz_rows_ref, xyz_cols_ref, feat_ref, w1_ref, b1_ref,
                w2_ref, b2_ref, out_ref):
    xr = xyz_rows_ref[0]
    xc = xyz_cols_ref[0]
    sqr = jnp.sum(xr * xr, axis=1, keepdims=True)
    sqc = jnp.sum(xc * xc, axis=0, keepdims=True)
    dot = jax.lax.dot_general(
        xr, xc, (((1,), (0,)), ((), ())),
        preferred_element_type=jnp.float32,
        precision=jax.lax.Precision.DEFAULT)
    d2 = jnp.maximum(sqr + sqc - 2.0 * dot, 0.0)

    nzero = jnp.sum((d2 == 0.0).astype(jnp.int32), axis=1, keepdims=True)
    steps = K - nzero

    def body(i, v):
        vn = jnp.min(jnp.where(d2 > v, d2, jnp.inf), axis=1, keepdims=True)
        adv = (i + 1) <= steps
        return jnp.where(adv, vn, v)

    vk = jax.lax.fori_loop(0, K, body, jnp.zeros((TG, 1), jnp.float32))
    mask = (d2 <= vk).astype(jnp.float32)

    feat = feat_ref[0]
    featsq = feat * feat
    pooled_sq = jax.lax.dot_general(
        mask, featsq, (((1,), (0,)), ((), ())),
        preferred_element_type=jnp.float32,
        precision=jax.lax.Precision.DEFAULT)
    pooled = jnp.sqrt(pooled_sq)

    h = jax.lax.dot_general(
        pooled, w1_ref[...], (((1,), (0,)), ((), ())),
        preferred_element_type=jnp.float32,
        precision=jax.lax.Precision.DEFAULT) + b1_ref[...]
    h = 0.5 * h * (1.0 + jax.lax.erf(h * jnp.float32(0.7071067811865476)))
    out = jax.lax.dot_general(
        h, w2_ref[...], (((1,), (0,)), ((), ())),
        preferred_element_type=jnp.float32,
        precision=jax.lax.Precision.DEFAULT) + b2_ref[...]
    out_ref[0] = out


@jax.jit
def kernel(xyz, feat, W1, b1, W2, b2):
    xyzT = jnp.transpose(xyz, (0, 2, 1))
    w1t = W1.T
    w2t = W2.T
    b1r = b1.reshape(1, OUT_C)
    b2r = b2.reshape(1, OUT_C)

    grid = (B, G // TG)
    return pl.pallas_call(
        _lnp_kernel,
        grid=grid,
        in_specs=[
            pl.BlockSpec((1, TG, 3), lambda b, i: (b, i, 0)),
            pl.BlockSpec((1, 3, G), lambda b, i: (b, 0, 0)),
            pl.BlockSpec((1, G, IN_C), lambda b, i: (b, 0, 0)),
            pl.BlockSpec((IN_C, OUT_C), lambda b, i: (0, 0)),
            pl.BlockSpec((1, OUT_C), lambda b, i: (0, 0)),
            pl.BlockSpec((OUT_C, OUT_C), lambda b, i: (0, 0)),
            pl.BlockSpec((1, OUT_C), lambda b, i: (0, 0)),
        ],
        out_specs=pl.BlockSpec((1, TG, OUT_C), lambda b, i: (b, i, 0)),
        out_shape=jax.ShapeDtypeStruct((B, G, OUT_C), jnp.float32),
    )(xyz, xyzT, feat, w1t, b1r, w2t, b2r)


# speedup vs baseline: 31.2540x; 1.1701x over previous
import jax
import jax.numpy as jnp
from jax.experimental import pallas as pl

B, G, IN_C, OUT_C, K = 8, 2048, 256, 256, 16
TG = 512


def _lnp_kernel(xyz_rows_ref, xyz_cols_ref, feat_ref, w1_ref, b1_ref,
                w2_ref, b2_ref, out_ref):
    xr = xyz_rows_ref[0]
    xc = xyz_cols_ref[0]
    sqr = jnp.sum(xr * xr, axis=1, keepdims=True)
    sqc = jnp.sum(xc * xc, axis=0, keepdims=True)
    dot = jax.lax.dot_general(
        xr, xc, (((1,), (0,)), ((), ())),
        preferred_element_type=jnp.float32,
        precision=jax.lax.Precision.DEFAULT)
    pre = sqr + sqc - 2.0 * dot
    d2 = jnp.maximum(pre, 0.0)

    nzero = jnp.sum((pre <= 0.0).astype(jnp.int32), axis=1, keepdims=True)
    steps = K - nzero

    def body(i, v):
        vn = jnp.min(jnp.where(d2 > v, d2, jnp.inf), axis=1, keepdims=True)
        adv = (i + 1) <= steps
        return jnp.where(adv, vn, v)

    vk = jax.lax.fori_loop(0, K, body, jnp.zeros((TG, 1), jnp.float32))
    mask = (d2 <= vk).astype(jnp.float32)

    feat = feat_ref[0]
    featsq = feat * feat
    pooled_sq = jax.lax.dot_general(
        mask, featsq, (((1,), (0,)), ((), ())),
        preferred_element_type=jnp.float32,
        precision=jax.lax.Precision.DEFAULT)
    pooled = jnp.sqrt(pooled_sq)

    h = jax.lax.dot_general(
        pooled, w1_ref[...], (((1,), (0,)), ((), ())),
        preferred_element_type=jnp.float32,
        precision=jax.lax.Precision.DEFAULT) + b1_ref[...]
    h = 0.5 * h * (1.0 + jax.lax.erf(h * jnp.float32(0.7071067811865476)))
    out = jax.lax.dot_general(
        h, w2_ref[...], (((1,), (0,)), ((), ())),
        preferred_element_type=jnp.float32,
        precision=jax.lax.Precision.DEFAULT) + b2_ref[...]
    out_ref[0] = out


@jax.jit
def kernel(xyz, feat, W1, b1, W2, b2):
    xyzT = jnp.transpose(xyz, (0, 2, 1))
    w1t = W1.T
    w2t = W2.T
    b1r = b1.reshape(1, OUT_C)
    b2r = b2.reshape(1, OUT_C)

    grid = (B, G // TG)
    return pl.pallas_call(
        _lnp_kernel,
        grid=grid,
        in_specs=[
            pl.BlockSpec((1, TG, 3), lambda b, i: (b, i, 0)),
            pl.BlockSpec((1, 3, G), lambda b, i: (b, 0, 0)),
            pl.BlockSpec((1, G, IN_C), lambda b, i: (b, 0, 0)),
            pl.BlockSpec((IN_C, OUT_C), lambda b, i: (0, 0)),
            pl.BlockSpec((1, OUT_C), lambda b, i: (0, 0)),
            pl.BlockSpec((OUT_C, OUT_C), lambda b, i: (0, 0)),
            pl.BlockSpec((1, OUT_C), lambda b, i: (0, 0)),
        ],
        out_specs=pl.BlockSpec((1, TG, OUT_C), lambda b, i: (b, i, 0)),
        out_shape=jax.ShapeDtypeStruct((B, G, OUT_C), jnp.float32),
    )(xyz, xyzT, feat, w1t, b1r, w2t, b2r)


# speedup vs baseline: 33.3333x; 1.0665x over previous
import jax
import jax.numpy as jnp
from jax.experimental import pallas as pl

B, G, IN_C, OUT_C, K = 8, 2048, 256, 256, 16
TG = 1024


def _lnp_kernel(xyz_rows_ref, xyz_cols_ref, feat_ref, w1_ref, b1_ref,
                w2_ref, b2_ref, out_ref):
    xr = xyz_rows_ref[0]
    xc = xyz_cols_ref[0]
    sqr = jnp.sum(xr * xr, axis=1, keepdims=True)
    sqc = jnp.sum(xc * xc, axis=0, keepdims=True)
    dot = jax.lax.dot_general(
        xr, xc, (((1,), (0,)), ((), ())),
        preferred_element_type=jnp.float32,
        precision=jax.lax.Precision.DEFAULT)
    pre = sqr + sqc - 2.0 * dot
    d2 = jnp.maximum(pre, 0.0)

    nzero = jnp.sum((pre <= 0.0).astype(jnp.int32), axis=1, keepdims=True)
    steps = K - nzero

    def body(i, v):
        vn = jnp.min(jnp.where(d2 > v, d2, jnp.inf), axis=1, keepdims=True)
        adv = (i + 1) <= steps
        return jnp.where(adv, vn, v)

    vk = jax.lax.fori_loop(0, K, body, jnp.zeros((TG, 1), jnp.float32))
    mask = (d2 <= vk).astype(jnp.float32)

    feat = feat_ref[0]
    featsq = feat * feat
    pooled_sq = jax.lax.dot_general(
        mask, featsq, (((1,), (0,)), ((), ())),
        preferred_element_type=jnp.float32,
        precision=jax.lax.Precision.DEFAULT)
    pooled = jnp.sqrt(pooled_sq)

    h = jax.lax.dot_general(
        pooled, w1_ref[...], (((1,), (0,)), ((), ())),
        preferred_element_type=jnp.float32,
        precision=jax.lax.Precision.DEFAULT) + b1_ref[...]
    h = 0.5 * h * (1.0 + jax.lax.erf(h * jnp.float32(0.7071067811865476)))
    out = jax.lax.dot_general(
        h, w2_ref[...], (((1,), (0,)), ((), ())),
        preferred_element_type=jnp.float32,
        precision=jax.lax.Precision.DEFAULT) + b2_ref[...]
    out_ref[0] = out


@jax.jit
def kernel(xyz, feat, W1, b1, W2, b2):
    xyzT = jnp.transpose(xyz, (0, 2, 1))
    w1t = W1.T
    w2t = W2.T
    b1r = b1.reshape(1, OUT_C)
    b2r = b2.reshape(1, OUT_C)

    grid = (B, G // TG)
    return pl.pallas_call(
        _lnp_kernel,
        grid=grid,
        in_specs=[
            pl.BlockSpec((1, TG, 3), lambda b, i: (b, i, 0)),
            pl.BlockSpec((1, 3, G), lambda b, i: (b, 0, 0)),
            pl.BlockSpec((1, G, IN_C), lambda b, i: (b, 0, 0)),
            pl.BlockSpec((IN_C, OUT_C), lambda b, i: (0, 0)),
            pl.BlockSpec((1, OUT_C), lambda b, i: (0, 0)),
            pl.BlockSpec((OUT_C, OUT_C), lambda b, i: (0, 0)),
            pl.BlockSpec((1, OUT_C), lambda b, i: (0, 0)),
        ],
        out_specs=pl.BlockSpec((1, TG, OUT_C), lambda b, i: (b, i, 0)),
        out_shape=jax.ShapeDtypeStruct((B, G, OUT_C), jnp.float32),
    )(xyz, xyzT, feat, w1t, b1r, w2t, b2r)


# speedup vs baseline: 34.2672x; 1.0280x over previous
import jax
import jax.numpy as jnp
from jax.experimental import pallas as pl

B, G, IN_C, OUT_C, K = 8, 2048, 256, 256, 16
TG = 2048


def _lnp_kernel(xyz_rows_ref, xyz_cols_ref, feat_ref, w1_ref, b1_ref,
                w2_ref, b2_ref, out_ref):
    xr = xyz_rows_ref[0]
    xc = xyz_cols_ref[0]
    sqr = jnp.sum(xr * xr, axis=1, keepdims=True)
    sqc = jnp.sum(xc * xc, axis=0, keepdims=True)
    dot = jax.lax.dot_general(
        xr, xc, (((1,), (0,)), ((), ())),
        preferred_element_type=jnp.float32,
        precision=jax.lax.Precision.DEFAULT)
    pre = sqr + sqc - 2.0 * dot
    d2 = jnp.maximum(pre, 0.0)

    nzero = jnp.sum((pre <= 0.0).astype(jnp.int32), axis=1, keepdims=True)
    steps = K - nzero

    def body(i, v):
        vn = jnp.min(jnp.where(d2 > v, d2, jnp.inf), axis=1, keepdims=True)
        adv = (i + 1) <= steps
        return jnp.where(adv, vn, v)

    vk = jax.lax.fori_loop(0, K, body, jnp.zeros((TG, 1), jnp.float32))
    mask = (d2 <= vk).astype(jnp.float32)

    feat = feat_ref[0]
    featsq = feat * feat
    pooled_sq = jax.lax.dot_general(
        mask, featsq, (((1,), (0,)), ((), ())),
        preferred_element_type=jnp.float32,
        precision=jax.lax.Precision.DEFAULT)
    pooled = jnp.sqrt(pooled_sq)

    h = jax.lax.dot_general(
        pooled, w1_ref[...], (((1,), (0,)), ((), ())),
        preferred_element_type=jnp.float32,
        precision=jax.lax.Precision.DEFAULT) + b1_ref[...]
    h = 0.5 * h * (1.0 + jax.lax.erf(h * jnp.float32(0.7071067811865476)))
    out = jax.lax.dot_general(
        h, w2_ref[...], (((1,), (0,)), ((), ())),
        preferred_element_type=jnp.float32,
        precision=jax.lax.Precision.DEFAULT) + b2_ref[...]
    out_ref[0] = out


@jax.jit
def kernel(xyz, feat, W1, b1, W2, b2):
    xyzT = jnp.transpose(xyz, (0, 2, 1))
    w1t = W1.T
    w2t = W2.T
    b1r = b1.reshape(1, OUT_C)
    b2r = b2.reshape(1, OUT_C)

    grid = (B, G // TG)
    return pl.pallas_call(
        _lnp_kernel,
        grid=grid,
        in_specs=[
            pl.BlockSpec((1, TG, 3), lambda b, i: (b, i, 0)),
            pl.BlockSpec((1, 3, G), lambda b, i: (b, 0, 0)),
            pl.BlockSpec((1, G, IN_C), lambda b, i: (b, 0, 0)),
            pl.BlockSpec((IN_C, OUT_C), lambda b, i: (0, 0)),
            pl.BlockSpec((1, OUT_C), lambda b, i: (0, 0)),
            pl.BlockSpec((OUT_C, OUT_C), lambda b, i: (0, 0)),
            pl.BlockSpec((1, OUT_C), lambda b, i: (0, 0)),
        ],
        out_specs=pl.BlockSpec((1, TG, OUT_C), lambda b, i: (b, i, 0)),
        out_shape=jax.ShapeDtypeStruct((B, G, OUT_C), jnp.float32),
    )(xyz, xyzT, feat, w1t, b1r, w2t, b2r)


# speedup vs baseline: 36.0061x; 1.0507x over previous
import jax
import jax.numpy as jnp
from jax.experimental import pallas as pl

B, G, IN_C, OUT_C, K = 8, 2048, 256, 256, 16
TG = 2048


def _lnp_kernel(xyz_rows_ref, xyz_cols_ref, feat_ref, w1_ref, b1_ref,
                w2_ref, b2_ref, out_ref):
    xr = xyz_rows_ref[0]
    xc = xyz_cols_ref[0]
    sqr = jnp.sum(xr * xr, axis=1, keepdims=True)
    sqc = jnp.sum(xc * xc, axis=0, keepdims=True)
    dot = jax.lax.dot_general(
        xr, xc, (((1,), (0,)), ((), ())),
        preferred_element_type=jnp.float32,
        precision=jax.lax.Precision.DEFAULT)
    iota = jax.lax.broadcasted_iota(jnp.int32, (TG, G), 1).astype(jnp.float32)
    d2 = jnp.maximum(sqr + sqc - 2.0 * dot, 0.0) + iota * jnp.float32(2.0 ** -34)

    def body(_, v):
        return jnp.min(jnp.where(d2 > v, d2, jnp.inf), axis=1, keepdims=True)

    vk = jax.lax.fori_loop(0, K, body,
                           jnp.full((TG, 1), -jnp.inf, dtype=jnp.float32))
    mask = (d2 <= vk).astype(jnp.bfloat16)

    feat = feat_ref[0]
    featsq = (feat * feat).astype(jnp.bfloat16)
    pooled_sq = jax.lax.dot_general(
        mask, featsq, (((1,), (0,)), ((), ())),
        preferred_element_type=jnp.float32,
        precision=jax.lax.Precision.DEFAULT)
    pooled = jnp.sqrt(pooled_sq)

    h = jax.lax.dot_general(
        pooled, w1_ref[...], (((1,), (0,)), ((), ())),
        preferred_element_type=jnp.float32,
        precision=jax.lax.Precision.DEFAULT) + b1_ref[...]
    h = 0.5 * h * (1.0 + jax.lax.erf(h * jnp.float32(0.7071067811865476)))
    out = jax.lax.dot_general(
        h, w2_ref[...], (((1,), (0,)), ((), ())),
        preferred_element_type=jnp.float32,
        precision=jax.lax.Precision.DEFAULT) + b2_ref[...]
    out_ref[0] = out


@jax.jit
def kernel(xyz, feat, W1, b1, W2, b2):
    xyzT = jnp.transpose(xyz, (0, 2, 1))
    w1t = W1.T
    w2t = W2.T
    b1r = b1.reshape(1, OUT_C)
    b2r = b2.reshape(1, OUT_C)

    grid = (B, G // TG)
    return pl.pallas_call(
        _lnp_kernel,
        grid=grid,
        in_specs=[
            pl.BlockSpec((1, TG, 3), lambda b, i: (b, i, 0)),
            pl.BlockSpec((1, 3, G), lambda b, i: (b, 0, 0)),
            pl.BlockSpec((1, G, IN_C), lambda b, i: (b, 0, 0)),
            pl.BlockSpec((IN_C, OUT_C), lambda b, i: (0, 0)),
            pl.BlockSpec((1, OUT_C), lambda b, i: (0, 0)),
            pl.BlockSpec((OUT_C, OUT_C), lambda b, i: (0, 0)),
            pl.BlockSpec((1, OUT_C), lambda b, i: (0, 0)),
        ],
        out_specs=pl.BlockSpec((1, TG, OUT_C), lambda b, i: (b, i, 0)),
        out_shape=jax.ShapeDtypeStruct((B, G, OUT_C), jnp.float32),
    )(xyz, xyzT, feat, w1t, b1r, w2t, b2r)


# speedup vs baseline: 37.5283x; 1.0423x over previous
import jax
import jax.numpy as jnp
from jax.experimental import pallas as pl

B, G, IN_C, OUT_C, K = 8, 2048, 256, 256, 16
TG = 2048


def _lnp_kernel(xyz_rows_ref, xyz_cols_ref, feat_ref, w1_ref, b1_ref,
                w2_ref, b2_ref, out_ref):
    xr = xyz_rows_ref[0]
    xc = xyz_cols_ref[0]
    sqr = jnp.sum(xr * xr, axis=1, keepdims=True)
    sqc = jnp.sum(xc * xc, axis=0, keepdims=True)
    dot = jax.lax.dot_general(
        xr, xc, (((1,), (0,)), ((), ())),
        preferred_element_type=jnp.float32,
        precision=jax.lax.Precision.DEFAULT)
    iota = jax.lax.broadcasted_iota(jnp.int32, (TG, G), 1).astype(jnp.float32)
    d2 = jnp.maximum(sqr + sqc - 2.0 * dot, 0.0) + iota * jnp.float32(2.0 ** -34)

    def body(_, v):
        return jnp.min(jnp.where(d2 > v, d2, jnp.inf), axis=1, keepdims=True)

    v1 = jnp.min(d2, axis=1, keepdims=True)
    vk = jax.lax.fori_loop(0, K - 1, body, v1)
    mask = (d2 <= vk).astype(jnp.bfloat16)

    feat = feat_ref[0]
    featsq = (feat * feat).astype(jnp.bfloat16)
    pooled_sq = jax.lax.dot_general(
        mask, featsq, (((1,), (0,)), ((), ())),
        preferred_element_type=jnp.float32,
        precision=jax.lax.Precision.DEFAULT)
    pooled = jnp.sqrt(pooled_sq)

    h = jax.lax.dot_general(
        pooled, w1_ref[...], (((1,), (0,)), ((), ())),
        preferred_element_type=jnp.float32,
        precision=jax.lax.Precision.DEFAULT) + b1_ref[...]
    h = 0.5 * h * (1.0 + jax.lax.erf(h * jnp.float32(0.7071067811865476)))
    out = jax.lax.dot_general(
        h, w2_ref[...], (((1,), (0,)), ((), ())),
        preferred_element_type=jnp.float32,
        precision=jax.lax.Precision.DEFAULT) + b2_ref[...]
    out_ref[0] = out


@jax.jit
def kernel(xyz, feat, W1, b1, W2, b2):
    xyzT = jnp.transpose(xyz, (0, 2, 1))
    w1t = W1.T
    w2t = W2.T
    b1r = b1.reshape(1, OUT_C)
    b2r = b2.reshape(1, OUT_C)

    grid = (B, G // TG)
    return pl.pallas_call(
        _lnp_kernel,
        grid=grid,
        in_specs=[
            pl.BlockSpec((1, TG, 3), lambda b, i: (b, i, 0)),
            pl.BlockSpec((1, 3, G), lambda b, i: (b, 0, 0)),
            pl.BlockSpec((1, G, IN_C), lambda b, i: (b, 0, 0)),
            pl.BlockSpec((IN_C, OUT_C), lambda b, i: (0, 0)),
            pl.BlockSpec((1, OUT_C), lambda b, i: (0, 0)),
            pl.BlockSpec((OUT_C, OUT_C), lambda b, i: (0, 0)),
            pl.BlockSpec((1, OUT_C), lambda b, i: (0, 0)),
        ],
        out_specs=pl.BlockSpec((1, TG, OUT_C), lambda b, i: (b, i, 0)),
        out_shape=jax.ShapeDtypeStruct((B, G, OUT_C), jnp.float32),
    )(xyz, xyzT, feat, w1t, b1r, w2t, b2r)
